# baseline (speedup 1.0000x reference)
import dataclasses
import functools

import jax
import jax.numpy as jnp
from jax.experimental import pallas as pl
from jax.experimental.pallas import tpu as pltpu
from jax.experimental.pallas import tpu_sc as plsc

D_MODEL = 4096
NUM_EXPERTS = 64
TOP_K = 8
BT = 1024
SC_BLK = 64


def _matmul_kernel(x_ref, w_ref, logits_ref):
    logits_ref[...] = jax.lax.dot_general(
        x_ref[...], w_ref[...], (((1,), (1,)), ((), ())),
        preferred_element_type=jnp.float32)


def _gather16(v, idx):
    return jax.lax.gather(
        v, idx[:, None],
        jax.lax.GatherDimensionNumbers(
            offset_dims=(), collapsed_slice_dims=(0,), start_index_map=(0,)),
        slice_sizes=(1,),
        mode=jax.lax.GatherScatterMode.PROMISE_IN_BOUNDS)


def _merge_top8(kA, vA, kB, vB, iota, mask8):
    shift = jnp.maximum(iota - 8, 0)
    kBs = _gather16(kB, shift)
    vBs = _gather16(vB, shift)
    kC = jnp.where(mask8, kA, kBs)
    vC = jnp.where(mask8, vA, vBs)
    return plsc.sort_key_val(kC, vC, descending=True)


def _sc_topk_call(logits):
    T = logits.shape[0]
    mesh = plsc.VectorSubcoreMesh(core_axis_name="core",
                                  subcore_axis_name="subcore")

    cp = pltpu.CompilerParams()
    if "needs_layout_passes" in pltpu.CompilerParams.__dataclass_fields__:
        cp = dataclasses.replace(cp, needs_layout_passes=False)

    @pl.kernel(out_type=(jax.ShapeDtypeStruct((T, 16), jnp.float32),
                         jax.ShapeDtypeStruct((T, 16), jnp.int32)),
               mesh=mesh, scratch_types=[], compiler_params=cp)
    def _k(lg_hbm, g_hbm, i_hbm):
        def body(lg_vmem, g_vmem, i_vmem):
            @pl.loop(0, SC_BLK)
            def _(t):
                iota = jax.lax.iota(jnp.int32, 16)
                mask8 = iota < 8
                sk, sv = [], []
                for c in range(4):
                    kc = lg_vmem.at[t, pl.ds(16 * c, 16)][...]
                    k_s, v_s = plsc.sort_key_val(kc, iota + 16 * c,
                                                 descending=True)
                    sk.append(k_s)
                    sv.append(v_s)
                kab, vab = _merge_top8(sk[0], sv[0], sk[1], sv[1], iota, mask8)
                kcd, vcd = _merge_top8(sk[2], sv[2], sk[3], sv[3], iota, mask8)
                kf, vf = _merge_top8(kab, vab, kcd, vcd, iota, mask8)
                mx = _gather16(kf, jnp.zeros((16,), jnp.int32))
                e = jnp.where(mask8, jnp.exp(kf - mx), 0.0)
                ssum = _gather16(plsc.cumsum(e),
                                 jnp.full((16,), 15, jnp.int32))
                g = e / ssum
                g_vmem.at[t, pl.ds(0, 16)][...] = g
                i_vmem.at[t, pl.ds(0, 16)][...] = vf

        pltpu.emit_pipeline(
            body,
            grid=(T // SC_BLK,),
            in_specs=[pl.BlockSpec((SC_BLK, NUM_EXPERTS), lambda i: (i, 0))],
            out_specs=[pl.BlockSpec((SC_BLK, 16), lambda i: (i, 0)),
                       pl.BlockSpec((SC_BLK, 16), lambda i: (i, 0))],
            core_axis_name=("core", "subcore"),
            dimension_semantics=(pltpu.PARALLEL,),
        )(lg_hbm, g_hbm, i_hbm)

    return _k(logits)


def kernel(x, W_gate, W_noise):
    B, N, D = x.shape
    T = B * N
    xf = x.reshape(T, D)
    logits = pl.pallas_call(
        _matmul_kernel,
        grid=(T // BT,),
        in_specs=[
            pl.BlockSpec((BT, D), lambda i: (i, 0)),
            pl.BlockSpec((NUM_EXPERTS, D), lambda i: (0, 0)),
        ],
        out_specs=pl.BlockSpec((BT, NUM_EXPERTS), lambda i: (i, 0)),
        out_shape=jax.ShapeDtypeStruct((T, NUM_EXPERTS), jnp.float32),
        compiler_params=pltpu.CompilerParams(
            dimension_semantics=("arbitrary",)),
    )(xf, W_gate)
    g16, i16 = _sc_topk_call(logits)
    return (g16[:, :TOP_K].reshape(B, N, TOP_K),
            i16[:, :TOP_K].reshape(B, N, TOP_K))


# speedup vs baseline: 1.7556x; 1.7556x over previous
import jax
import jax.numpy as jnp
from jax.experimental import pallas as pl
from jax.experimental.pallas import tpu as pltpu

D_MODEL = 4096
NUM_EXPERTS = 64
TOP_K = 8


def _gating_kernel(x_ref, w_ref, gates_ref, idx_ref):
    x = x_ref[...]
    w = w_ref[...]
    logits = jax.lax.dot_general(
        w, x, (((1,), (1,)), ((), ())),
        preferred_element_type=jnp.float32)
    iota = jax.lax.broadcasted_iota(jnp.int32, logits.shape, 0)
    work = logits
    vals, idxs = [], []
    for _ in range(TOP_K):
        m = jnp.max(work, axis=0, keepdims=True)
        hit = jnp.min(jnp.where(work == m, iota, NUM_EXPERTS),
                      axis=0, keepdims=True)
        vals.append(m)
        idxs.append(hit)
        work = jnp.where(iota == hit, -jnp.inf, work)
    v = jnp.concatenate(vals, axis=0)
    ix = jnp.concatenate(idxs, axis=0)
    e = jnp.exp(v - v[:1])
    gates_ref[...] = e / jnp.sum(e, axis=0, keepdims=True)
    idx_ref[...] = ix


def kernel(x, W_gate, W_noise):
    B, N, D = x.shape
    T = B * N
    xf = x.reshape(T, D)
    BT = 1024
    gates_t, idx_t = pl.pallas_call(
        _gating_kernel,
        grid=(T // BT,),
        in_specs=[
            pl.BlockSpec((BT, D), lambda i: (i, 0)),
            pl.BlockSpec((NUM_EXPERTS, D), lambda i: (0, 0)),
        ],
        out_specs=[
            pl.BlockSpec((TOP_K, BT), lambda i: (0, i)),
            pl.BlockSpec((TOP_K, BT), lambda i: (0, i)),
        ],
        out_shape=[
            jax.ShapeDtypeStruct((TOP_K, T), jnp.float32),
            jax.ShapeDtypeStruct((TOP_K, T), jnp.int32),
        ],
        compiler_params=pltpu.CompilerParams(
            dimension_semantics=("arbitrary",)),
    )(xf, W_gate)
    gates = gates_t.T.reshape(B, N, TOP_K)
    idx = idx_t.T.reshape(B, N, TOP_K)
    return gates, idx
